# baseline (speedup 1.0000x reference)
import functools

import jax
import jax.numpy as jnp
from jax import lax
from jax.experimental import pallas as pl
from jax.experimental.pallas import tpu as pltpu
from jax.experimental.pallas import tpu_sc as plsc


def _make_gather(n_idx: int, chunk: int):
    info = plsc.get_sparse_core_info()
    nc, ns, nl = info.num_cores, info.num_subcores, info.num_lanes
    nw = nc * ns
    assert n_idx % nw == 0
    per_w = n_idx // nw
    assert per_w % chunk == 0 and chunk % (4 * nl) == 0
    n_chunks = per_w // chunk
    assert n_chunks % 2 == 0 and n_chunks >= 4
    n2 = n_chunks // 2
    groups = chunk // nl

    mesh = plsc.VectorSubcoreMesh(core_axis_name="c", subcore_axis_name="s")

    @functools.partial(
        pl.kernel,
        mesh=mesh,
        out_type=jax.ShapeDtypeStruct((n_idx // 4, 128), jnp.float32),
        compiler_params=pltpu.CompilerParams(needs_layout_passes=False),
        scratch_types=[
            pltpu.VMEM((per_w,), jnp.int32),
            pltpu.VMEM((chunk,), jnp.int32),
            pltpu.VMEM((chunk,), jnp.int32),
            pltpu.VMEM((chunk, 128), jnp.float32),
            pltpu.VMEM((chunk, 128), jnp.float32),
            pltpu.VMEM((chunk // 4, 128), jnp.float32),
            pltpu.VMEM((chunk // 4, 128), jnp.float32),
            pltpu.SemaphoreType.DMA,
            pltpu.SemaphoreType.DMA,
            pltpu.SemaphoreType.DMA,
            pltpu.SemaphoreType.DMA,
        ],
    )
    def gather_kernel(table_hbm, idx_hbm, out_hbm, idx_v, rb0, rb1, r0, r1,
                      ob0, ob1, sg0, sg1, so0, so1):
        wid = lax.axis_index("s") * nc + lax.axis_index("c")
        base = wid * per_w
        pltpu.sync_copy(idx_hbm.at[pl.ds(base, per_w)], idx_v)
        lanes = lax.iota(jnp.int32, nl)

        def build_ridx(c, rbuf):
            def one(t, carry):
                v = idx_v[pl.ds(c * chunk + t * nl, nl)]
                rbuf[pl.ds(t * nl, nl)] = lax.shift_right_logical(v, 2)
                return carry
            lax.fori_loop(0, groups, one, 0, unroll=4)

        def fire_gather(rbuf, buf, sem):
            pltpu.async_copy(table_hbm.at[rbuf], buf, sem)

        def wait_gather(buf, sem):
            pltpu.make_async_copy(table_hbm.at[rb0], buf, sem).wait()

        def fire_out(c, obuf, sem):
            off = pl.multiple_of((base + c * chunk) // 4, 8)
            pltpu.async_copy(obuf, out_hbm.at[pl.ds(off, chunk // 4)], sem)

        def wait_out(obuf, sem):
            pltpu.make_async_copy(obuf, out_hbm.at[pl.ds(0, chunk // 4)],
                                  sem).wait()

        def extract(c, rows, obuf):
            def one(t, carry):
                v = idx_v[pl.ds(c * chunk + t * nl, nl)]
                q32 = lax.shift_left((v & 3), 5)
                jv = t * nl + lanes
                rv = lax.shift_right_logical(jv, 2)
                cb = lax.shift_left((jv & 3), 5)
                for d in range(32):
                    val = plsc.load_gather(rows, [jv, q32 + d])
                    plsc.store_scatter(obuf, [rv, cb + d], val)
                return carry
            lax.fori_loop(0, groups, one, 0, unroll=2)

        build_ridx(0, rb0)
        fire_gather(rb0, r0, sg0)
        build_ridx(1, rb1)
        fire_gather(rb1, r1, sg1)

        def step(i, carry):
            g = 2 * i
            wait_gather(r0, sg0)

            @pl.when(i > 0)
            def _():
                wait_out(ob0, so0)

            extract(g, r0, ob0)
            fire_out(g, ob0, so0)

            @pl.when(i < n2 - 1)
            def _():
                build_ridx(g + 2, rb0)
                fire_gather(rb0, r0, sg0)

            wait_gather(r1, sg1)

            @pl.when(i > 0)
            def _():
                wait_out(ob1, so1)

            extract(g + 1, r1, ob1)
            fire_out(g + 1, ob1, so1)

            @pl.when(i < n2 - 1)
            def _():
                build_ridx(g + 3, rb1)
                fire_gather(rb1, r1, sg1)

            return carry

        lax.fori_loop(0, n2, step, 0)
        wait_out(ob0, so0)
        wait_out(ob1, so1)

    return gather_kernel


def kernel(input_seq, weight):
    b, s = input_seq.shape
    vocab, d = weight.shape
    assert d == 32
    idx = input_seq.reshape(-1).astype(jnp.int32)
    table = weight.reshape(vocab // 4, 128)
    out = _make_gather(b * s, chunk=256)(table, idx)
    return out.reshape(b, s, d)


# speedup vs baseline: 1.8848x; 1.8848x over previous
import functools

import jax
import jax.numpy as jnp
from jax import lax
from jax.experimental import pallas as pl
from jax.experimental.pallas import tpu as pltpu
from jax.experimental.pallas import tpu_sc as plsc


def _make_gather(n_idx: int, d: int, chunk: int):
    info = plsc.get_sparse_core_info()
    nc, ns = info.num_cores, info.num_subcores
    nw = nc * ns
    assert n_idx % nw == 0
    per_w = n_idx // nw
    assert per_w % chunk == 0 and chunk % 8 == 0
    n_chunks = per_w // chunk
    assert n_chunks % 2 == 0 and n_chunks >= 4

    mesh = plsc.VectorSubcoreMesh(core_axis_name="c", subcore_axis_name="s")

    @functools.partial(
        pl.kernel,
        mesh=mesh,
        out_type=jax.ShapeDtypeStruct((n_idx, d), jnp.float32),
        compiler_params=pltpu.CompilerParams(use_tc_tiling_on_sc=False),
        scratch_types=[
            pltpu.VMEM((per_w,), jnp.int32),
            pltpu.VMEM((chunk, d), jnp.float32),
            pltpu.VMEM((chunk, d), jnp.float32),
            pltpu.SemaphoreType.DMA,
            pltpu.SemaphoreType.DMA,
            pltpu.SemaphoreType.DMA,
            pltpu.SemaphoreType.DMA,
        ],
    )
    def gather_kernel(table_hbm, idx_hbm, out_hbm, idx_v, r0, r1, sg0, sg1,
                      so0, so1):
        wid = lax.axis_index("s") * nc + lax.axis_index("c")
        base = wid * per_w
        pltpu.sync_copy(idx_hbm.at[pl.ds(base, per_w)], idx_v)

        def fire_gather(c, buf, sem):
            pltpu.async_copy(table_hbm.at[idx_v.at[pl.ds(c * chunk, chunk)]],
                             buf, sem)

        def wait_gather(buf, sem):
            pltpu.make_async_copy(
                table_hbm.at[idx_v.at[pl.ds(0, chunk)]], buf, sem).wait()

        def fire_out(c, buf, sem):
            pltpu.async_copy(buf, out_hbm.at[pl.ds(base + c * chunk, chunk)],
                             sem)

        def wait_out(buf, sem):
            pltpu.make_async_copy(buf, out_hbm.at[pl.ds(0, chunk)],
                                  sem).wait()

        fire_gather(0, r0, sg0)
        fire_gather(1, r1, sg1)

        def step(i, carry):
            g = 2 * i
            wait_gather(r0, sg0)
            fire_out(g, r0, so0)
            wait_gather(r1, sg1)
            fire_out(g + 1, r1, so1)
            wait_out(r0, so0)
            fire_gather(g + 2, r0, sg0)
            wait_out(r1, so1)
            fire_gather(g + 3, r1, sg1)
            return carry

        lax.fori_loop(0, n_chunks // 2 - 1, step, 0)

        g_last = n_chunks - 2
        wait_gather(r0, sg0)
        fire_out(g_last, r0, so0)
        wait_gather(r1, sg1)
        fire_out(g_last + 1, r1, so1)
        wait_out(r0, so0)
        wait_out(r1, so1)

    return gather_kernel


def kernel(input_seq, weight):
    b, s = input_seq.shape
    vocab, d = weight.shape
    idx = input_seq.reshape(-1).astype(jnp.int32)
    out = _make_gather(b * s, d, chunk=1280)(weight, idx)
    return out.reshape(b, s, d)
